# baseline (speedup 1.0000x reference)
import jax
import jax.numpy as jnp
from jax import lax
from jax.experimental import pallas as pl
from jax.experimental.pallas import tpu as pltpu
from jax.experimental.pallas import tpu_sc as plsc

_BATCH = 16384
_D = 32
_PACK = 4
_PW = _PACK * _D
_NC = 2
_NS = 16
_NW = _NC * _NS
_BPW = _BATCH // _NW
_CHUNK = 128
_NCHUNK = _BPW // _CHUNK
_GPC = _CHUNK // 16
_L = 16


def _mf_body(uid_hbm, iid_hbm, u4_hbm, i4_hbm, uoff_hbm, ioff_hbm,
             p4_hbm, q4_hbm, ub_hbm, ib_hbm, out_hbm,
             uidx, iidx, u4, i4, uoff, ioff,
             prow, qrow, bu_v, bi_v, out_v, sem0, sem1, semb):
    wid = lax.axis_index("s") * _NC + lax.axis_index("c")
    sems = (sem0, sem1)
    pltpu.sync_copy(uid_hbm.at[wid], uidx)
    pltpu.sync_copy(iid_hbm.at[wid], iidx)
    pltpu.sync_copy(u4_hbm.at[wid], u4)
    pltpu.sync_copy(i4_hbm.at[wid], i4)
    pltpu.sync_copy(uoff_hbm.at[wid], uoff)
    pltpu.sync_copy(ioff_hbm.at[wid], ioff)

    bias_cps = []
    for c in range(_NCHUNK):
        sl = pl.ds(c * _CHUNK, _CHUNK)
        bias_cps.append(pltpu.async_copy(ub_hbm.at[uidx.at[c]], bu_v.at[sl], semb))
        bias_cps.append(pltpu.async_copy(ib_hbm.at[iidx.at[c]], bi_v.at[sl], semb))

    def fire(c):
        buf = c % 2
        return [
            pltpu.async_copy(p4_hbm.at[u4.at[c]], prow.at[buf], sems[buf]),
            pltpu.async_copy(q4_hbm.at[i4.at[c]], qrow.at[buf], sems[buf]),
        ]

    lane = lax.broadcasted_iota(jnp.int32, (_L,), 0)
    pending = fire(0)
    for c in range(_NCHUNK):
        nxt = fire(c + 1) if c + 1 < _NCHUNK else []
        for cp in pending:
            cp.wait()
        pending = nxt
        if c == 0:
            for cp in bias_cps:
                cp.wait()
        buf = c % 2
        pc = prow.at[buf]
        qc = qrow.at[buf]

        def group(g, carry):
            gb = g * _L
            acc = bu_v[pl.ds(c * _CHUNK + gb, _L)] + bi_v[pl.ds(c * _CHUNK + gb, _L)]
            row_idx = lane + gb
            ucol = uoff[c, pl.ds(gb, _L)]
            icol = ioff[c, pl.ds(gb, _L)]
            for k in range(_D):
                pv = plsc.load_gather(pc, [row_idx, ucol + k])
                qv = plsc.load_gather(qc, [row_idx, icol + k])
                acc = acc + pv * qv
            out_v[pl.ds(c * _CHUNK + gb, _L)] = acc
            return carry

        lax.fori_loop(0, _GPC, group, 0)

    pltpu.sync_copy(out_v, out_hbm.at[pl.ds(wid * _BPW, _BPW)])


@jax.jit
def _mf(uid3, iid3, u4_3, i4_3, uoff3, ioff3, P4, Q4, ub, ib):
    mesh = plsc.VectorSubcoreMesh(core_axis_name="c", subcore_axis_name="s")
    return pl.kernel(
        _mf_body,
        mesh=mesh,
        compiler_params=pltpu.CompilerParams(needs_layout_passes=False),
        out_type=jax.ShapeDtypeStruct((_BATCH,), jnp.float32),
        scratch_types=[
            pltpu.VMEM((_NCHUNK, _CHUNK), jnp.int32),
            pltpu.VMEM((_NCHUNK, _CHUNK), jnp.int32),
            pltpu.VMEM((_NCHUNK, _CHUNK), jnp.int32),
            pltpu.VMEM((_NCHUNK, _CHUNK), jnp.int32),
            pltpu.VMEM((_NCHUNK, _CHUNK), jnp.int32),
            pltpu.VMEM((_NCHUNK, _CHUNK), jnp.int32),
            pltpu.VMEM((2, _CHUNK, _PW), jnp.float32),
            pltpu.VMEM((2, _CHUNK, _PW), jnp.float32),
            pltpu.VMEM((_BPW,), jnp.float32),
            pltpu.VMEM((_BPW,), jnp.float32),
            pltpu.VMEM((_BPW,), jnp.float32),
            pltpu.SemaphoreType.DMA,
            pltpu.SemaphoreType.DMA,
            pltpu.SemaphoreType.DMA,
        ],
    )(uid3, iid3, u4_3, i4_3, uoff3, ioff3, P4, Q4, ub, ib)


def kernel(user_id, item_id, P, Q, user_bias, item_bias):
    shape3 = (_NW, _NCHUNK, _CHUNK)
    uid3 = user_id.reshape(shape3)
    iid3 = item_id.reshape(shape3)
    u4_3 = (uid3 // _PACK).astype(jnp.int32)
    i4_3 = (iid3 // _PACK).astype(jnp.int32)
    uoff3 = ((uid3 % _PACK) * _D).astype(jnp.int32)
    ioff3 = ((iid3 % _PACK) * _D).astype(jnp.int32)
    P4 = P.reshape(-1, _PW)
    Q4 = Q.reshape(-1, _PW)
    ub = user_bias.reshape(-1)
    ib = item_bias.reshape(-1)
    return _mf(uid3, iid3, u4_3, i4_3, uoff3, ioff3, P4, Q4, ub, ib)
